# baseline (speedup 1.0000x reference)
import functools

import jax
import jax.numpy as jnp
from jax import lax
from jax.experimental import pallas as pl
from jax.experimental.pallas import tpu as pltpu
from jax.experimental.pallas import tpu_sc as plsc

N = 100000
E = 1600000
BLK = 2000

RPN = 3136
NPAD = 32 * RPN
CH_D = 8000
CH_A = 4000
HALF = 50000

_MESH = dict(core_axis_name="c", subcore_axis_name="s")
_CP = pltpu.CompilerParams(needs_layout_passes=False)


@functools.partial(
    pl.kernel,
    out_type=jax.ShapeDtypeStruct((NPAD,), jnp.float32),
    mesh=plsc.VectorSubcoreMesh(**_MESH),
    scratch_types=[
        pltpu.VMEM((CH_D,), jnp.int32),
        pltpu.VMEM((RPN,), jnp.float32),
    ],
    compiler_params=_CP,
)
def _sc_deg(dst_hbm, out_hbm, dchunk, acc):
    c = lax.axis_index("c")
    s = lax.axis_index("s")
    w = s * 2 + c
    lo = w * RPN
    zero16 = jnp.zeros((16,), jnp.float32)
    ones16 = jnp.ones((16,), jnp.float32)

    def zero_body(j, carry):
        acc[pl.ds(j * 16, 16)] = zero16
        return carry

    lax.fori_loop(0, RPN // 16, zero_body, 0)

    def chunk_body(kk, carry):
        pltpu.sync_copy(dst_hbm.at[pl.ds(kk * CH_D, CH_D)], dchunk)

        def vec_body(j, c2):
            dv = dchunk[pl.ds(j * 16, 16)]
            ldv = dv - lo
            m = (ldv >= 0) & (ldv < RPN)
            plsc.addupdate_scatter(acc, [ldv], ones16, mask=m)
            return c2

        lax.fori_loop(0, CH_D // 16, vec_body, 0)
        return carry

    lax.fori_loop(0, E // CH_D, chunk_body, 0)
    pltpu.sync_copy(acc, out_hbm.at[pl.ds(lo, RPN)])


@functools.partial(
    pl.kernel,
    out_type=jax.ShapeDtypeStruct((2 * NPAD,), jnp.float32),
    mesh=plsc.VectorSubcoreMesh(**_MESH),
    scratch_types=[
        pltpu.VMEM((CH_A,), jnp.int32),
        pltpu.VMEM((CH_A,), jnp.int32),
        pltpu.VMEM((2 * HALF,), jnp.float32),
        pltpu.VMEM((2 * RPN,), jnp.float32),
    ],
    compiler_params=_CP,
)
def _sc_agg2(u_hbm, src_hbm, dst_hbm, out_hbm, schunk, dchunk, table, acc):
    c = lax.axis_index("c")
    s = lax.axis_index("s")
    w = s * 2 + c
    lo = w * RPN
    zero16 = jnp.zeros((16,), jnp.float32)

    def zero_body(j, carry):
        acc[pl.ds(j * 16, 16)] = zero16
        return carry

    lax.fori_loop(0, 2 * RPN // 16, zero_body, 0)

    for p in range(2):
        pltpu.sync_copy(u_hbm.at[pl.ds(p * 2 * HALF, 2 * HALF)], table)

        def chunk_body(kk, carry):
            pltpu.sync_copy(src_hbm.at[pl.ds(kk * CH_A, CH_A)], schunk)
            pltpu.sync_copy(dst_hbm.at[pl.ds(kk * CH_A, CH_A)], dchunk)

            def vec_body(j, c2):
                dv = dchunk[pl.ds(j * 16, 16)]
                sv = schunk[pl.ds(j * 16, 16)]
                ldv = dv - lo
                lsv = sv - p * HALF
                m = ((ldv >= 0) & (ldv < RPN)
                     & (lsv >= 0) & (lsv < HALF))
                g0 = plsc.load_gather(table, [2 * lsv], mask=m)
                g1 = plsc.load_gather(table, [2 * lsv + 1], mask=m)
                i0 = 2 * ldv
                plsc.addupdate_scatter(acc, [i0], g0, mask=m)
                plsc.addupdate_scatter(acc, [i0 + 1], g1, mask=m)
                return c2

            lax.fori_loop(0, CH_A // 16, vec_body, 0)
            return carry

        lax.fori_loop(0, E // CH_A, chunk_body, 0)

    pltpu.sync_copy(acc, out_hbm.at[pl.ds(2 * lo, 2 * RPN)])


CAPR = 256
CH_AGG = 2048
CH_R = 4000
ECAP = 1724416
NFL = 32 * 16


@functools.partial(
    pl.kernel,
    out_type=(jax.ShapeDtypeStruct((32 * ECAP,), jnp.int32),
              jax.ShapeDtypeStruct((32 * ECAP,), jnp.int32),
              jax.ShapeDtypeStruct((NFL,), jnp.int32)),
    mesh=plsc.VectorSubcoreMesh(**_MESH),
    scratch_types=[
        pltpu.VMEM((CH_R,), jnp.int32),
        pltpu.VMEM((CH_R,), jnp.int32),
        pltpu.VMEM((CAPR,), jnp.int32),
        pltpu.VMEM((CAPR,), jnp.int32),
        pltpu.VMEM((16,), jnp.int32),
        pltpu.SMEM((1,), jnp.int32),
    ],
    compiler_params=_CP,
)
def _sc_route(src_hbm, dst_hbm, bsrc_hbm, bldst_hbm, cnt_hbm,
              schunk, dchunk, msrc, mldst, cbuf, cur):
    c = lax.axis_index("c")
    s = lax.axis_index("s")
    w = s * 2 + c
    lo = w * RPN
    obase = w * ECAP
    zero16 = jnp.zeros((16,), jnp.int32)
    neg16 = jnp.full((16,), -1, jnp.int32)
    zcnt = jnp.zeros((16,), jnp.int32)

    for t in range(CAPR // 16):
        msrc[pl.ds(t * 16, 16)] = zero16
        mldst[pl.ds(t * 16, 16)] = neg16
    cur[0] = 0

    def flush():
        cur0 = pl.multiple_of(cur[0], CAPR)
        pltpu.sync_copy(msrc, bsrc_hbm.at[pl.ds(obase + cur0, CAPR)])
        pltpu.sync_copy(mldst, bldst_hbm.at[pl.ds(obase + cur0, CAPR)])
        for t in range(CAPR // 16):
            msrc[pl.ds(t * 16, 16)] = zero16
            mldst[pl.ds(t * 16, 16)] = neg16
        cur[0] = cur0 + CAPR

    def chunk_body(kk, cntv):
        pltpu.sync_copy(src_hbm.at[pl.ds(kk * CH_R, CH_R)], schunk)
        pltpu.sync_copy(dst_hbm.at[pl.ds(kk * CH_R, CH_R)], dchunk)

        def vec_body(j, cv):
            dv = dchunk[pl.ds(j * 16, 16)]
            sv = schunk[pl.ds(j * 16, 16)]
            ldv = dv - lo
            m = (ldv >= 0) & (ldv < RPN)
            mi = m.astype(jnp.int32)
            inc = plsc.cumsum(mi)
            pos = cv + inc - 1
            plsc.store_scatter(mldst, [pos], ldv, mask=m)
            plsc.store_scatter(msrc, [pos], sv, mask=m)
            pc = plsc.all_reduce_population_count(m)
            cv2 = cv + pc
            do_flush = jnp.any(cv2 > CAPR - 16)
            pl.when(do_flush)(flush)
            return jnp.where(do_flush, zcnt, cv2)

        return lax.fori_loop(0, CH_R // 16, vec_body, cntv)

    lax.fori_loop(0, E // CH_R, chunk_body, zcnt)
    flush()
    padn = ((CH_AGG - cur[0] % CH_AGG) % CH_AGG) // CAPR
    for t in range(CH_AGG // CAPR):
        pl.when(t < padn)(flush)
    cbuf[pl.ds(0, 16)] = jnp.full((16,), cur[0], jnp.int32)
    pltpu.sync_copy(cbuf, cnt_hbm.at[pl.ds(w * 16, 16)])


def _make_sc_aggw(F, SBN, NSB, CAPC, CH_C):
    assert NSB * SBN == RPN

    @functools.partial(
        pl.kernel,
        out_type=jax.ShapeDtypeStruct((NPAD, F), jnp.float32),
        mesh=plsc.VectorSubcoreMesh(**_MESH),
        scratch_types=[
            pltpu.VMEM((SBN + 1, F), jnp.float32),
            pltpu.VMEM((CAPC, F), jnp.float32),
            pltpu.VMEM((CH_C,), jnp.int32),
            pltpu.VMEM((CH_C,), jnp.int32),
            pltpu.VMEM((CAPC,), jnp.int32),
            pltpu.VMEM((CAPC,), jnp.int32),
            pltpu.VMEM((16,), jnp.int32),
            pltpu.SemaphoreType.DMA,
        ],
        compiler_params=_CP,
    )
    def k(y_hbm, bsrc_hbm, bldst_hbm, cnt_hbm, out_hbm,
          acc, rowbuf, lsrc, lldst, gsrc, gldst, cntb, gsem):
        c = lax.axis_index("c")
        s = lax.axis_index("s")
        w = s * 2 + c
        obase = w * ECAP
        zero16f = jnp.zeros((16,), jnp.float32)
        zero16 = jnp.zeros((16,), jnp.int32)
        neg16 = jnp.full((16,), -1, jnp.int32)
        zcnt = jnp.zeros((16,), jnp.int32)

        pltpu.sync_copy(cnt_hbm.at[pl.ds(w * 16, 16)], cntb)
        nch = cntb[pl.ds(0, 16)][0] // CH_C

        for t in range(CAPC // 16):
            gsrc[pl.ds(t * 16, 16)] = zero16
            gldst[pl.ds(t * 16, 16)] = neg16

        def do_batch():
            pltpu.async_copy(y_hbm.at[gsrc], rowbuf, gsem).wait()

            def add_body(g, carry):
                ov = gldst[pl.ds(g * 16, 16)]
                ov = jnp.where(ov >= 0, ov, SBN)
                for l in range(16):
                    o = ov[l]
                    rb = g * 16 + l

                    def col_body(kq, c3):
                        acc[o, pl.ds(kq * 16, 16)] = (
                            acc[o, pl.ds(kq * 16, 16)]
                            + rowbuf[rb, pl.ds(kq * 16, 16)])
                        return c3

                    lax.fori_loop(0, F // 16, col_body, 0)
                return carry

            lax.fori_loop(0, CAPC // 16, add_body, 0)
            for t in range(CAPC // 16):
                gsrc[pl.ds(t * 16, 16)] = zero16
                gldst[pl.ds(t * 16, 16)] = neg16

        def sb_body(sb, carry):
            slo = sb * SBN

            def zero_body(j, c4):
                def zcol(kq, c3):
                    acc[j, pl.ds(kq * 16, 16)] = zero16f
                    return c3

                lax.fori_loop(0, F // 16, zcol, 0)
                return c4

            lax.fori_loop(0, SBN + 1, zero_body, 0)

            def chunk_body(kk, cntv):
                pltpu.sync_copy(bsrc_hbm.at[pl.ds(obase + kk * CH_C, CH_C)],
                                lsrc)
                pltpu.sync_copy(bldst_hbm.at[pl.ds(obase + kk * CH_C, CH_C)],
                                lldst)

                def vec_body(j, cv):
                    ldv = lldst[pl.ds(j * 16, 16)] - slo
                    sv = lsrc[pl.ds(j * 16, 16)]
                    m = (ldv >= 0) & (ldv < SBN)
                    mi = m.astype(jnp.int32)
                    inc = plsc.cumsum(mi)
                    pos = cv + inc - 1
                    plsc.store_scatter(gldst, [pos], ldv, mask=m)
                    plsc.store_scatter(gsrc, [pos], sv, mask=m)
                    pc = plsc.all_reduce_population_count(m)
                    cv2 = cv + pc
                    do_b = jnp.any(cv2 > CAPC - 16)
                    pl.when(do_b)(do_batch)
                    return jnp.where(do_b, zcnt, cv2)

                return lax.fori_loop(0, CH_C // 16, vec_body, cntv)

            lax.fori_loop(0, nch, chunk_body, zcnt)
            do_batch()

            pltpu.sync_copy(acc.at[pl.ds(0, SBN)],
                            out_hbm.at[pl.ds(w * RPN + sb * SBN, SBN)])
            return carry

        lax.fori_loop(0, NSB, sb_body, 0)

    return k


_sc_agg256 = _make_sc_aggw(256, 224, 14, 128, 2048)
_sc_agg128 = _make_sc_aggw(128, 448, 7, 128, 2048)


def _row_spec(f):
    return pl.BlockSpec((BLK, f), lambda i: (i, 0))


def _full_spec(shape):
    return pl.BlockSpec(shape, lambda i: tuple(0 for _ in shape))


def _tc_call(body, out_shapes, in_arrays, in_specs, out_specs):
    return pl.pallas_call(
        body,
        grid=(N // BLK,),
        in_specs=in_specs,
        out_specs=out_specs,
        out_shape=out_shapes,
    )(*in_arrays)


def _prep_body(d_ref, x_ref, r_ref, u0_ref):
    deg = d_ref[...] + 1.0
    r = lax.rsqrt(jnp.maximum(deg, 1e-12))
    r_ref[...] = r
    u0_ref[...] = r * x_ref[...]


def _tcA_body(a_ref, u0_ref, r_ref, W1_ref, b1_ref, u1_ref):
    r = r_ref[...]
    sH = r * (a_ref[...] + u0_ref[...])
    h = jnp.dot(sH, W1_ref[...], preferred_element_type=jnp.float32)
    h = jnp.maximum(h + b1_ref[...], 0.0)
    u1_ref[...] = r * h


def _tcB_body(agg1_ref, u1_ref, r_ref, W2_ref, b2_ref, W3_ref, g_ref):
    r = r_ref[...]
    sH = r * (agg1_ref[...] + u1_ref[...])
    h = jnp.dot(sH, W2_ref[...], preferred_element_type=jnp.float32)
    h = jnp.maximum(h + b2_ref[...], 0.0)
    g_ref[...] = jnp.dot(r * h, W3_ref[...], preferred_element_type=jnp.float32)


def _tcC_body(aggg_ref, g_ref, r_ref, b3_ref, Wp1_ref, bp1_ref, Wp2_ref,
              bp2_ref, o_ref):
    r = r_ref[...]
    h3 = jnp.maximum(r * (aggg_ref[...] + g_ref[...]) + b3_ref[...], 0.0)
    p = jnp.dot(h3, Wp1_ref[...], preferred_element_type=jnp.float32)
    p = jnp.maximum(p + bp1_ref[...], 0.0)
    o = jnp.dot(p, Wp2_ref[...], preferred_element_type=jnp.float32)
    o_ref[...] = jax.nn.sigmoid(o + bp2_ref[...])


def kernel(x, edge_index, W1, b1, W2, b2, W3, b3, Wp1, bp1, Wp2, bp2):
    src = edge_index[0].astype(jnp.int32)
    dst = edge_index[1].astype(jnp.int32)

    bsrc, bldst, ecnt = _sc_route(src, dst)
    deg = _sc_deg(dst)[:N].reshape(N, 1)
    r, u0 = _tc_call(
        _prep_body,
        (jax.ShapeDtypeStruct((N, 1), jnp.float32),
         jax.ShapeDtypeStruct((N, 2), jnp.float32)),
        (deg, x),
        [_row_spec(1), _row_spec(2)],
        (_row_spec(1), _row_spec(2)),
    )

    agg0 = _sc_agg2(u0.reshape(2 * N), src, dst).reshape(NPAD, 2)[:N]
    u1 = _tc_call(
        _tcA_body, jax.ShapeDtypeStruct((N, 256), jnp.float32),
        (agg0, u0, r, W1, b1.reshape(1, 256)),
        [_row_spec(2), _row_spec(2), _row_spec(1), _full_spec((2, 256)),
         _full_spec((1, 256))],
        _row_spec(256),
    )

    agg1 = _sc_agg256(u1, bsrc, bldst, ecnt)[:N]
    g = _tc_call(
        _tcB_body, jax.ShapeDtypeStruct((N, 128), jnp.float32),
        (agg1, u1, r, W2, b2.reshape(1, 256), W3),
        [_row_spec(256), _row_spec(256), _row_spec(1), _full_spec((256, 256)),
         _full_spec((1, 256)), _full_spec((256, 128))],
        _row_spec(128),
    )

    agg2 = _sc_agg128(g, bsrc, bldst, ecnt)[:N]
    out = _tc_call(
        _tcC_body, jax.ShapeDtypeStruct((N, 1), jnp.float32),
        (agg2, g, r, b3.reshape(1, 128), Wp1, bp1.reshape(1, 32), Wp2,
         bp2.reshape(1, 1)),
        [_row_spec(128), _row_spec(128), _row_spec(1), _full_spec((1, 128)),
         _full_spec((128, 32)), _full_spec((1, 32)), _full_spec((32, 1)),
         _full_spec((1, 1))],
        _row_spec(1),
    )
    return out


# speedup vs baseline: 1.0059x; 1.0059x over previous
import functools

import jax
import jax.numpy as jnp
from jax import lax
from jax.experimental import pallas as pl
from jax.experimental.pallas import tpu as pltpu
from jax.experimental.pallas import tpu_sc as plsc

N = 100000
E = 1600000
BLK = 2000

RPN = 3136
NPAD = 32 * RPN
CH_D = 8000
CH_A = 4000
HALF = 50000

_MESH = dict(core_axis_name="c", subcore_axis_name="s")
_CP = pltpu.CompilerParams(needs_layout_passes=False)


@functools.partial(
    pl.kernel,
    out_type=jax.ShapeDtypeStruct((NPAD,), jnp.float32),
    mesh=plsc.VectorSubcoreMesh(**_MESH),
    scratch_types=[
        pltpu.VMEM((CH_D,), jnp.int32),
        pltpu.VMEM((RPN,), jnp.float32),
    ],
    compiler_params=_CP,
)
def _sc_deg(dst_hbm, out_hbm, dchunk, acc):
    c = lax.axis_index("c")
    s = lax.axis_index("s")
    w = s * 2 + c
    lo = w * RPN
    zero16 = jnp.zeros((16,), jnp.float32)
    ones16 = jnp.ones((16,), jnp.float32)

    def zero_body(j, carry):
        acc[pl.ds(j * 16, 16)] = zero16
        return carry

    lax.fori_loop(0, RPN // 16, zero_body, 0)

    def chunk_body(kk, carry):
        pltpu.sync_copy(dst_hbm.at[pl.ds(kk * CH_D, CH_D)], dchunk)

        def vec_body(j, c2):
            dv = dchunk[pl.ds(j * 16, 16)]
            ldv = dv - lo
            m = (ldv >= 0) & (ldv < RPN)
            plsc.addupdate_scatter(acc, [ldv], ones16, mask=m)
            return c2

        lax.fori_loop(0, CH_D // 16, vec_body, 0)
        return carry

    lax.fori_loop(0, E // CH_D, chunk_body, 0)
    pltpu.sync_copy(acc, out_hbm.at[pl.ds(lo, RPN)])


@functools.partial(
    pl.kernel,
    out_type=jax.ShapeDtypeStruct((2 * NPAD,), jnp.float32),
    mesh=plsc.VectorSubcoreMesh(**_MESH),
    scratch_types=[
        pltpu.VMEM((CH_A,), jnp.int32),
        pltpu.VMEM((CH_A,), jnp.int32),
        pltpu.VMEM((2 * HALF,), jnp.float32),
        pltpu.VMEM((2 * RPN,), jnp.float32),
    ],
    compiler_params=_CP,
)
def _sc_agg2(u_hbm, src_hbm, dst_hbm, out_hbm, schunk, dchunk, table, acc):
    c = lax.axis_index("c")
    s = lax.axis_index("s")
    w = s * 2 + c
    lo = w * RPN
    zero16 = jnp.zeros((16,), jnp.float32)

    def zero_body(j, carry):
        acc[pl.ds(j * 16, 16)] = zero16
        return carry

    lax.fori_loop(0, 2 * RPN // 16, zero_body, 0)

    for p in range(2):
        pltpu.sync_copy(u_hbm.at[pl.ds(p * 2 * HALF, 2 * HALF)], table)

        def chunk_body(kk, carry):
            pltpu.sync_copy(src_hbm.at[pl.ds(kk * CH_A, CH_A)], schunk)
            pltpu.sync_copy(dst_hbm.at[pl.ds(kk * CH_A, CH_A)], dchunk)

            def vec_body(j, c2):
                dv = dchunk[pl.ds(j * 16, 16)]
                sv = schunk[pl.ds(j * 16, 16)]
                ldv = dv - lo
                lsv = sv - p * HALF
                m = ((ldv >= 0) & (ldv < RPN)
                     & (lsv >= 0) & (lsv < HALF))
                g0 = plsc.load_gather(table, [2 * lsv], mask=m)
                g1 = plsc.load_gather(table, [2 * lsv + 1], mask=m)
                i0 = 2 * ldv
                plsc.addupdate_scatter(acc, [i0], g0, mask=m)
                plsc.addupdate_scatter(acc, [i0 + 1], g1, mask=m)
                return c2

            lax.fori_loop(0, CH_A // 16, vec_body, 0)
            return carry

        lax.fori_loop(0, E // CH_A, chunk_body, 0)

    pltpu.sync_copy(acc, out_hbm.at[pl.ds(2 * lo, 2 * RPN)])


CAPR = 256
CH_AGG = 2048
CH_R = 4000
ECAP = 1724416
NFL = 32 * 16


@functools.partial(
    pl.kernel,
    out_type=(jax.ShapeDtypeStruct((32 * ECAP,), jnp.int32),
              jax.ShapeDtypeStruct((32 * ECAP,), jnp.int32),
              jax.ShapeDtypeStruct((NFL,), jnp.int32)),
    mesh=plsc.VectorSubcoreMesh(**_MESH),
    scratch_types=[
        pltpu.VMEM((CH_R,), jnp.int32),
        pltpu.VMEM((CH_R,), jnp.int32),
        pltpu.VMEM((CAPR,), jnp.int32),
        pltpu.VMEM((CAPR,), jnp.int32),
        pltpu.VMEM((16,), jnp.int32),
        pltpu.SMEM((1,), jnp.int32),
    ],
    compiler_params=_CP,
)
def _sc_route(src_hbm, dst_hbm, bsrc_hbm, bldst_hbm, cnt_hbm,
              schunk, dchunk, msrc, mldst, cbuf, cur):
    c = lax.axis_index("c")
    s = lax.axis_index("s")
    w = s * 2 + c
    lo = w * RPN
    obase = w * ECAP
    zero16 = jnp.zeros((16,), jnp.int32)
    neg16 = jnp.full((16,), -1, jnp.int32)
    zcnt = jnp.zeros((16,), jnp.int32)

    for t in range(CAPR // 16):
        msrc[pl.ds(t * 16, 16)] = zero16
        mldst[pl.ds(t * 16, 16)] = neg16
    cur[0] = 0

    def flush():
        cur0 = pl.multiple_of(cur[0], CAPR)
        pltpu.sync_copy(msrc, bsrc_hbm.at[pl.ds(obase + cur0, CAPR)])
        pltpu.sync_copy(mldst, bldst_hbm.at[pl.ds(obase + cur0, CAPR)])
        for t in range(CAPR // 16):
            msrc[pl.ds(t * 16, 16)] = zero16
            mldst[pl.ds(t * 16, 16)] = neg16
        cur[0] = cur0 + CAPR

    def chunk_body(kk, cntv):
        pltpu.sync_copy(src_hbm.at[pl.ds(kk * CH_R, CH_R)], schunk)
        pltpu.sync_copy(dst_hbm.at[pl.ds(kk * CH_R, CH_R)], dchunk)

        def vec_body(j, cv):
            dv = dchunk[pl.ds(j * 16, 16)]
            sv = schunk[pl.ds(j * 16, 16)]
            ldv = dv - lo
            m = (ldv >= 0) & (ldv < RPN)
            mi = m.astype(jnp.int32)
            inc = plsc.cumsum(mi)
            pos = cv + inc - 1
            plsc.store_scatter(mldst, [pos], ldv, mask=m)
            plsc.store_scatter(msrc, [pos], sv, mask=m)
            pc = plsc.all_reduce_population_count(m)
            cv2 = cv + pc
            do_flush = jnp.any(cv2 > CAPR - 16)
            pl.when(do_flush)(flush)
            return jnp.where(do_flush, zcnt, cv2)

        return lax.fori_loop(0, CH_R // 16, vec_body, cntv)

    lax.fori_loop(0, E // CH_R, chunk_body, zcnt)
    flush()
    padn = ((CH_AGG - cur[0] % CH_AGG) % CH_AGG) // CAPR
    for t in range(CH_AGG // CAPR):
        pl.when(t < padn)(flush)
    cbuf[pl.ds(0, 16)] = jnp.full((16,), cur[0], jnp.int32)
    pltpu.sync_copy(cbuf, cnt_hbm.at[pl.ds(w * 16, 16)])


def _make_sc_aggw(F, SBN, NSB, CAPC, CH_C):
    assert NSB * SBN == RPN

    @functools.partial(
        pl.kernel,
        out_type=jax.ShapeDtypeStruct((NPAD, F), jnp.float32),
        mesh=plsc.VectorSubcoreMesh(**_MESH),
        scratch_types=[
            pltpu.VMEM((SBN + 1, F), jnp.float32),
            pltpu.VMEM((CAPC, F), jnp.float32),
            pltpu.VMEM((CH_C,), jnp.int32),
            pltpu.VMEM((CH_C,), jnp.int32),
            pltpu.VMEM((CAPC,), jnp.int32),
            pltpu.VMEM((CAPC,), jnp.int32),
            pltpu.VMEM((16,), jnp.int32),
            pltpu.SemaphoreType.DMA,
        ],
        compiler_params=_CP,
    )
    def k(y_hbm, bsrc_hbm, bldst_hbm, cnt_hbm, out_hbm,
          acc, rowbuf, lsrc, lldst, gsrc, gldst, cntb, gsem):
        c = lax.axis_index("c")
        s = lax.axis_index("s")
        w = s * 2 + c
        obase = w * ECAP
        zero16f = jnp.zeros((16,), jnp.float32)
        zero16 = jnp.zeros((16,), jnp.int32)
        neg16 = jnp.full((16,), -1, jnp.int32)
        zcnt = jnp.zeros((16,), jnp.int32)

        pltpu.sync_copy(cnt_hbm.at[pl.ds(w * 16, 16)], cntb)
        nch = cntb[pl.ds(0, 16)][0] // CH_C

        for t in range(CAPC // 16):
            gsrc[pl.ds(t * 16, 16)] = zero16
            gldst[pl.ds(t * 16, 16)] = neg16

        def do_batch():
            pltpu.async_copy(y_hbm.at[gsrc], rowbuf, gsem).wait()

            def add_body(g, carry):
                ov = gldst[pl.ds(g * 16, 16)]
                ov = jnp.where(ov >= 0, ov, SBN)
                for l in range(16):
                    o = ov[l]
                    rb = g * 16 + l
                    for kq in range(F // 16):
                        acc[o, pl.ds(kq * 16, 16)] = (
                            acc[o, pl.ds(kq * 16, 16)]
                            + rowbuf[rb, pl.ds(kq * 16, 16)])
                return carry

            lax.fori_loop(0, CAPC // 16, add_body, 0)
            for t in range(CAPC // 16):
                gsrc[pl.ds(t * 16, 16)] = zero16
                gldst[pl.ds(t * 16, 16)] = neg16

        def sb_body(sb, carry):
            slo = sb * SBN

            def zero_body(j, c4):
                def zcol(kq, c3):
                    acc[j, pl.ds(kq * 16, 16)] = zero16f
                    return c3

                lax.fori_loop(0, F // 16, zcol, 0)
                return c4

            lax.fori_loop(0, SBN + 1, zero_body, 0)

            def chunk_body(kk, cntv):
                pltpu.sync_copy(bsrc_hbm.at[pl.ds(obase + kk * CH_C, CH_C)],
                                lsrc)
                pltpu.sync_copy(bldst_hbm.at[pl.ds(obase + kk * CH_C, CH_C)],
                                lldst)

                def vec_body(j, cv):
                    ldv = lldst[pl.ds(j * 16, 16)] - slo
                    sv = lsrc[pl.ds(j * 16, 16)]
                    m = (ldv >= 0) & (ldv < SBN)
                    mi = m.astype(jnp.int32)
                    inc = plsc.cumsum(mi)
                    pos = cv + inc - 1
                    plsc.store_scatter(gldst, [pos], ldv, mask=m)
                    plsc.store_scatter(gsrc, [pos], sv, mask=m)
                    pc = plsc.all_reduce_population_count(m)
                    cv2 = cv + pc
                    do_b = jnp.any(cv2 > CAPC - 16)
                    pl.when(do_b)(do_batch)
                    return jnp.where(do_b, zcnt, cv2)

                return lax.fori_loop(0, CH_C // 16, vec_body, cntv)

            lax.fori_loop(0, nch, chunk_body, zcnt)
            do_batch()

            pltpu.sync_copy(acc.at[pl.ds(0, SBN)],
                            out_hbm.at[pl.ds(w * RPN + sb * SBN, SBN)])
            return carry

        lax.fori_loop(0, NSB, sb_body, 0)

    return k


_sc_agg256 = _make_sc_aggw(256, 224, 14, 128, 2048)
_sc_agg128 = _make_sc_aggw(128, 448, 7, 128, 2048)


def _row_spec(f):
    return pl.BlockSpec((BLK, f), lambda i: (i, 0))


def _full_spec(shape):
    return pl.BlockSpec(shape, lambda i: tuple(0 for _ in shape))


def _tc_call(body, out_shapes, in_arrays, in_specs, out_specs):
    return pl.pallas_call(
        body,
        grid=(N // BLK,),
        in_specs=in_specs,
        out_specs=out_specs,
        out_shape=out_shapes,
    )(*in_arrays)


def _prep_body(d_ref, x_ref, r_ref, u0_ref):
    deg = d_ref[...] + 1.0
    r = lax.rsqrt(jnp.maximum(deg, 1e-12))
    r_ref[...] = r
    u0_ref[...] = r * x_ref[...]


def _tcA_body(a_ref, u0_ref, r_ref, W1_ref, b1_ref, u1_ref):
    r = r_ref[...]
    sH = r * (a_ref[...] + u0_ref[...])
    h = jnp.dot(sH, W1_ref[...], preferred_element_type=jnp.float32)
    h = jnp.maximum(h + b1_ref[...], 0.0)
    u1_ref[...] = r * h


def _tcB_body(agg1_ref, u1_ref, r_ref, W2_ref, b2_ref, W3_ref, g_ref):
    r = r_ref[...]
    sH = r * (agg1_ref[...] + u1_ref[...])
    h = jnp.dot(sH, W2_ref[...], preferred_element_type=jnp.float32)
    h = jnp.maximum(h + b2_ref[...], 0.0)
    g_ref[...] = jnp.dot(r * h, W3_ref[...], preferred_element_type=jnp.float32)


def _tcC_body(aggg_ref, g_ref, r_ref, b3_ref, Wp1_ref, bp1_ref, Wp2_ref,
              bp2_ref, o_ref):
    r = r_ref[...]
    h3 = jnp.maximum(r * (aggg_ref[...] + g_ref[...]) + b3_ref[...], 0.0)
    p = jnp.dot(h3, Wp1_ref[...], preferred_element_type=jnp.float32)
    p = jnp.maximum(p + bp1_ref[...], 0.0)
    o = jnp.dot(p, Wp2_ref[...], preferred_element_type=jnp.float32)
    o_ref[...] = jax.nn.sigmoid(o + bp2_ref[...])


def kernel(x, edge_index, W1, b1, W2, b2, W3, b3, Wp1, bp1, Wp2, bp2):
    src = edge_index[0].astype(jnp.int32)
    dst = edge_index[1].astype(jnp.int32)

    bsrc, bldst, ecnt = _sc_route(src, dst)
    deg = _sc_deg(dst)[:N].reshape(N, 1)
    r, u0 = _tc_call(
        _prep_body,
        (jax.ShapeDtypeStruct((N, 1), jnp.float32),
         jax.ShapeDtypeStruct((N, 2), jnp.float32)),
        (deg, x),
        [_row_spec(1), _row_spec(2)],
        (_row_spec(1), _row_spec(2)),
    )

    agg0 = _sc_agg2(u0.reshape(2 * N), src, dst).reshape(NPAD, 2)[:N]
    u1 = _tc_call(
        _tcA_body, jax.ShapeDtypeStruct((N, 256), jnp.float32),
        (agg0, u0, r, W1, b1.reshape(1, 256)),
        [_row_spec(2), _row_spec(2), _row_spec(1), _full_spec((2, 256)),
         _full_spec((1, 256))],
        _row_spec(256),
    )

    agg1 = _sc_agg256(u1, bsrc, bldst, ecnt)[:N]
    g = _tc_call(
        _tcB_body, jax.ShapeDtypeStruct((N, 128), jnp.float32),
        (agg1, u1, r, W2, b2.reshape(1, 256), W3),
        [_row_spec(256), _row_spec(256), _row_spec(1), _full_spec((256, 256)),
         _full_spec((1, 256)), _full_spec((256, 128))],
        _row_spec(128),
    )

    agg2 = _sc_agg128(g, bsrc, bldst, ecnt)[:N]
    out = _tc_call(
        _tcC_body, jax.ShapeDtypeStruct((N, 1), jnp.float32),
        (agg2, g, r, b3.reshape(1, 128), Wp1, bp1.reshape(1, 32), Wp2,
         bp2.reshape(1, 1)),
        [_row_spec(128), _row_spec(128), _row_spec(1), _full_spec((1, 128)),
         _full_spec((128, 32)), _full_spec((1, 32)), _full_spec((32, 1)),
         _full_spec((1, 1))],
        _row_spec(1),
    )
    return out
